# baseline (speedup 1.0000x reference)
import functools
import jax
import jax.numpy as jnp
from jax.experimental import pallas as pl
from jax.experimental.pallas import tpu as pltpu

E = 16
D = 1024
H = 4096
O = 1024
B = 128
HBLK = 512
NHB = H // HBLK


def _gating_body(x_ref, gw_ref, gb_ref,
                 gates_ref, gates_t_ref, tidx_ref, load_ref, imp_ref):
    logits = jnp.dot(x_ref[...], gw_ref[...],
                     preferred_element_type=jnp.float32) + gb_ref[...]
    e_iota = jax.lax.broadcasted_iota(jnp.int32, (B, E), 1)
    m1 = jnp.max(logits, axis=1, keepdims=True)
    idx1 = jnp.min(jnp.where(logits == m1, e_iota, E), axis=1, keepdims=True)
    masked = jnp.where(e_iota == idx1, -jnp.inf, logits)
    m2 = jnp.max(masked, axis=1, keepdims=True)
    idx2 = jnp.min(jnp.where(masked == m2, e_iota, E), axis=1, keepdims=True)
    z = jnp.exp(m2 - m1)
    g1 = 1.0 / (1.0 + z)
    g2 = z / (1.0 + z)
    gates = (jnp.where(e_iota == idx1, g1, 0.0)
             + jnp.where(e_iota == idx2, g2, 0.0))
    gates_ref[...] = gates
    gates_t_ref[...] = gates.T
    tidx_ref[...] = jnp.concatenate([idx1, idx2], axis=1)
    s = jnp.sum(gates, axis=0, keepdims=True)
    load_ref[...] = s * (1.0 / B)
    imp_ref[...] = s


def _gating(x, gate_W, gate_b):
    return pl.pallas_call(
        _gating_body,
        out_shape=(
            jax.ShapeDtypeStruct((B, E), jnp.float32),
            jax.ShapeDtypeStruct((E, B), jnp.float32),
            jax.ShapeDtypeStruct((B, 2), jnp.int32),
            jax.ShapeDtypeStruct((1, E), jnp.float32),
            jax.ShapeDtypeStruct((1, E), jnp.float32),
        ),
    )(x, gate_W, gate_b.reshape(1, E))


def _moe_body(gates_t_ref, x_ref, w1_ref, b1_ref, w2_ref, b2_ref, out_ref):
    e = pl.program_id(0)
    hb = pl.program_id(1)

    @pl.when((e == 0) & (hb == 0))
    def _():
        out_ref[...] = jnp.zeros_like(out_ref)

    h = jnp.maximum(
        jnp.dot(x_ref[...], w1_ref[0], preferred_element_type=jnp.float32)
        + b1_ref[0], 0.0)
    part = jnp.dot(h, w2_ref[0], preferred_element_type=jnp.float32)
    g = gates_t_ref[0, 0, :].reshape(B, 1)

    @pl.when(hb == NHB - 1)
    def _():
        out_ref[...] += g * (part + b2_ref[0])

    @pl.when(hb != NHB - 1)
    def _():
        out_ref[...] += g * part


def _moe(gates_t, x, W1, b1, W2, b2):
    return pl.pallas_call(
        _moe_body,
        grid=(E, NHB),
        in_specs=[
            pl.BlockSpec((1, 1, B), lambda e, h: (e, 0, 0)),
            pl.BlockSpec((B, D), lambda e, h: (0, 0)),
            pl.BlockSpec((1, D, HBLK), lambda e, h: (e, 0, h)),
            pl.BlockSpec((1, 1, HBLK), lambda e, h: (e, 0, h)),
            pl.BlockSpec((1, HBLK, O), lambda e, h: (e, h, 0)),
            pl.BlockSpec((1, 1, O), lambda e, h: (e, 0, 0)),
        ],
        out_specs=pl.BlockSpec((B, O), lambda e, h: (0, 0)),
        out_shape=jax.ShapeDtypeStruct((B, O), jnp.float32),
    )(gates_t, x, W1, b1, W2, b2)


@jax.jit
def kernel(x, gate_W, gate_b, W1, b1, W2, b2):
    gates, gates_t, top_idx, load, imp = _gating(x, gate_W, gate_b)
    output = _moe(gates_t.reshape(E, 1, B), x, W1,
                  b1.reshape(E, 1, H), W2, b2.reshape(E, 1, O))
    return (output, gates, load.reshape(E), imp.reshape(E), top_idx)
